# baseline (speedup 1.0000x reference)
import functools

import jax
import jax.numpy as jnp
from jax import lax
from jax.experimental import pallas as pl
from jax.experimental.pallas import tpu as pltpu
from jax.experimental.pallas import tpu_sc as plsc

B = 16384
T = 26
TH = 13
DIM = 32
NUM_WORKERS = 32
IBLK = 128
BLOCKS_PER_WORKER = B // (NUM_WORKERS * IBLK)
HROWS = IBLK * TH

_mesh = plsc.VectorSubcoreMesh(core_axis_name="c", subcore_axis_name="s")


@functools.partial(
    pl.kernel,
    out_type=jax.ShapeDtypeStruct((T, 4, B // IBLK, 1024), jnp.float32),
    mesh=_mesh,
    scratch_types=[
        pltpu.VMEM((IBLK * T,), jnp.int32),
        pltpu.VMEM((HROWS,), jnp.int32),
        pltpu.VMEM((HROWS,), jnp.int32),
        pltpu.VMEM((HROWS, DIM), jnp.float32),
        pltpu.VMEM((HROWS, DIM), jnp.float32),
        pltpu.VMEM((4, 1024), jnp.float32),
        pltpu.VMEM((4, 1024), jnp.float32),
        pltpu.SemaphoreType.DMA,
        pltpu.SemaphoreType.DMA,
        pltpu.SemaphoreType.DMA,
        pltpu.SemaphoreType.DMA,
    ],
    compiler_params=pltpu.CompilerParams(
        use_tc_tiling_on_sc=False, needs_layout_passes=False
    ),
)
def _embed_sc(
    idx_hbm,
    tbl_hbm,
    out_hbm,
    idx_blk,
    ib0,
    ib1,
    rows0,
    rows1,
    ov0,
    ov1,
    g0,
    g1,
    w0,
    w1,
):
    wid = lax.axis_index("s") * 2 + lax.axis_index("c")
    iota = lax.iota(jnp.int32, 16)
    iota_t = iota * T

    def build_ib(h, ib):
        for jt in range(TH):
            vals = [
                plsc.load_gather(idx_blk, [iota_t + (lc * 16 * T + h * TH + jt)])
                for lc in range(IBLK // 16)
            ]
            for lc in range(IBLK // 16):
                ib[pl.ds(jt * IBLK + lc * 16, 16)] = vals[lc]

    def transpose_t(rows, jt, ov):
        rowbase = iota + jt * IBLK
        for tr in range(4):
            for s in range(8):
                col = jnp.full((16,), 8 * tr + s, jnp.int32)
                vals = [
                    plsc.load_gather(rows, [rowbase + lc * 16, col])
                    for lc in range(8)
                ]
                for lc in range(8):
                    ov[tr, pl.ds(s * 128 + lc * 16, 16)] = vals[lc]

    def wait_write(sem, ov):
        pltpu.make_async_copy(ov, out_hbm.at[0, :, 0], sem).wait()

    def do_block(bi, carry):
        blk = wid * BLOCKS_PER_WORKER + bi
        pltpu.sync_copy(idx_hbm.at[pl.ds(blk * IBLK * T, IBLK * T)], idx_blk)
        build_ib(0, ib0)
        pltpu.async_copy(tbl_hbm.at[ib0], rows0, g0)
        build_ib(1, ib1)
        pltpu.async_copy(tbl_hbm.at[ib1], rows1, g1)

        for h, rows, gsem in ((0, rows0, g0), (1, rows1, g1)):
            pltpu.make_async_copy(tbl_hbm.at[ib0], rows, gsem).wait()

            def pair(jt2, carry, h=h, rows=rows):
                te = 2 * jt2
                to = te + 1

                @pl.when(jnp.logical_or(bi > 0, jt2 > 0) if h == 0 else jt2 >= 0)
                def _():
                    wait_write(w0, ov0)

                transpose_t(rows, te, ov0)
                pltpu.async_copy(ov0, out_hbm.at[h * TH + te, :, blk], w0)

                @pl.when(
                    jnp.logical_or(bi > 0, jt2 > 0) if h == 0 else jt2 >= 0
                )
                def _():
                    wait_write(w1, ov1)

                transpose_t(rows, to, ov1)
                pltpu.async_copy(ov1, out_hbm.at[h * TH + to, :, blk], w1)
                return carry

            lax.fori_loop(0, TH // 2, pair, 0)
            wait_write(w0, ov0)
            transpose_t(rows, TH - 1, ov0)
            pltpu.async_copy(ov0, out_hbm.at[h * TH + TH - 1, :, blk], w0)
        return carry

    lax.fori_loop(0, BLOCKS_PER_WORKER, do_block, 0)
    wait_write(w0, ov0)
    wait_write(w1, ov1)


def kernel(idx, weight):
    idx_flat = idx.reshape(-1).astype(jnp.int32)
    out5 = _embed_sc(idx_flat, weight)
    out5 = out5.reshape(T, 4, B // IBLK, 8, 128)
    return out5.transpose(2, 4, 0, 1, 3).reshape(B, T, DIM)


# speedup vs baseline: 1.0145x; 1.0145x over previous
import functools

import jax
import jax.numpy as jnp
from jax import lax
from jax.experimental import pallas as pl
from jax.experimental.pallas import tpu as pltpu
from jax.experimental.pallas import tpu_sc as plsc

B = 16384
T = 26
TH = 13
DIM = 32
NUM_WORKERS = 32
IBLK = 128
BLOCKS_PER_WORKER = B // (NUM_WORKERS * IBLK)
HROWS = IBLK * TH

_mesh = plsc.VectorSubcoreMesh(core_axis_name="c", subcore_axis_name="s")


@functools.partial(
    pl.kernel,
    out_type=jax.ShapeDtypeStruct((T, 4, B // IBLK, 1024), jnp.float32),
    mesh=_mesh,
    scratch_types=[
        pltpu.VMEM((IBLK * T,), jnp.int32),
        pltpu.VMEM((HROWS,), jnp.int32),
        pltpu.VMEM((HROWS,), jnp.int32),
        pltpu.VMEM((HROWS, DIM), jnp.float32),
        pltpu.VMEM((HROWS, DIM), jnp.float32),
        pltpu.VMEM((4, 1024), jnp.float32),
        pltpu.VMEM((4, 1024), jnp.float32),
        pltpu.SemaphoreType.DMA,
        pltpu.SemaphoreType.DMA,
        pltpu.SemaphoreType.DMA,
        pltpu.SemaphoreType.DMA,
    ],
    compiler_params=pltpu.CompilerParams(
        use_tc_tiling_on_sc=False, needs_layout_passes=False
    ),
)
def _embed_sc(
    idx_hbm,
    tbl_hbm,
    out_hbm,
    idx_blk,
    ib0,
    ib1,
    rows0,
    rows1,
    ov0,
    ov1,
    g0,
    g1,
    w0,
    w1,
):
    wid = lax.axis_index("s") * 2 + lax.axis_index("c")
    iota = lax.iota(jnp.int32, 16)
    iota_t = iota * T

    def build_ib(h, ib):
        for jt in range(TH):
            vals = [
                plsc.load_gather(idx_blk, [iota_t + (lc * 16 * T + h * TH + jt)])
                for lc in range(IBLK // 16)
            ]
            for lc in range(IBLK // 16):
                ib[pl.ds(jt * IBLK + lc * 16, 16)] = vals[lc]

    def transpose_t(rows, jt, ov):
        rowbase = iota + jt * IBLK
        for tr in range(4):
            for sp in range(4):
                chunks = [(sp * 2 + sh, lc) for sh in range(2) for lc in range(8)]
                vals = [
                    plsc.load_gather(
                        rows,
                        [
                            rowbase + lc * 16,
                            jnp.full((16,), 8 * tr + s, jnp.int32),
                        ],
                    )
                    for s, lc in chunks
                ]
                for v, (s, lc) in zip(vals, chunks):
                    ov[tr, pl.ds(s * 128 + lc * 16, 16)] = v

    def wait_write(sem, ov):
        pltpu.make_async_copy(ov, out_hbm.at[0, :, 0], sem).wait()

    def do_block(bi, carry):
        blk = wid * BLOCKS_PER_WORKER + bi
        pltpu.sync_copy(idx_hbm.at[pl.ds(blk * IBLK * T, IBLK * T)], idx_blk)
        build_ib(0, ib0)
        pltpu.async_copy(tbl_hbm.at[ib0], rows0, g0)
        build_ib(1, ib1)
        pltpu.async_copy(tbl_hbm.at[ib1], rows1, g1)

        for h, rows, gsem in ((0, rows0, g0), (1, rows1, g1)):
            pltpu.make_async_copy(tbl_hbm.at[ib0], rows, gsem).wait()

            def pair(jt2, carry, h=h, rows=rows):
                te = 2 * jt2
                to = te + 1

                @pl.when(jnp.logical_or(bi > 0, jt2 > 0) if h == 0 else jt2 >= 0)
                def _():
                    wait_write(w0, ov0)

                transpose_t(rows, te, ov0)
                pltpu.async_copy(ov0, out_hbm.at[h * TH + te, :, blk], w0)

                @pl.when(
                    jnp.logical_or(bi > 0, jt2 > 0) if h == 0 else jt2 >= 0
                )
                def _():
                    wait_write(w1, ov1)

                transpose_t(rows, to, ov1)
                pltpu.async_copy(ov1, out_hbm.at[h * TH + to, :, blk], w1)
                return carry

            lax.fori_loop(0, TH // 2, pair, 0)
            wait_write(w0, ov0)
            transpose_t(rows, TH - 1, ov0)
            pltpu.async_copy(ov0, out_hbm.at[h * TH + TH - 1, :, blk], w0)
        return carry

    lax.fori_loop(0, BLOCKS_PER_WORKER, do_block, 0)
    wait_write(w0, ov0)
    wait_write(w1, ov1)


def kernel(idx, weight):
    idx_flat = idx.reshape(-1).astype(jnp.int32)
    out5 = _embed_sc(idx_flat, weight)
    out5 = out5.reshape(T, 4, B // IBLK, 8, 128)
    return out5.transpose(2, 4, 0, 1, 3).reshape(B, T, DIM)
